# Initial kernel scaffold; baseline (speedup 1.0000x reference)
import jax
import jax.numpy as jnp
from jax.experimental import pallas as pl


def kernel(edge_index, edge_vals, embeds):
    raise NotImplementedError("write your pallas kernel here")


# speedup vs baseline: 2.7813x; 2.7813x over previous
import jax
import jax.numpy as jnp
from jax import lax
from jax.experimental import pallas as pl
from jax.experimental.pallas import tpu as pltpu, tpu_sc as plsc

N_NODES = 10000
N_EDGES = 160000
D_FEAT = 256

NC = 2
NS = 16
DH = D_FEAT // NC
EPT = N_EDGES // NS
CHUNK = 80
NCHUNK = EPT // CHUNK
RPT = N_NODES // NS
RBLK = 125
NRB = RPT // RBLK


def _gcn_body(emb_hbm, col_hbm, row_hbm, vals_hbm, out_hbm,
              col_v, row_v, vals_v, rows_v, obuf, acc, sem):
    c = lax.axis_index("c")
    s = lax.axis_index("s")

    def _zero(i, _):
        for g in range(DH // 16):
            obuf[i, pl.ds(g * 16, 16)] = jnp.zeros((16,), jnp.float32)
        return 0
    lax.fori_loop(0, RBLK, _zero, 0)
    for b in range(NRB):
        pltpu.sync_copy(obuf, acc.at[pl.ds(s * RPT + b * RBLK, RBLK), :])
    plsc.subcore_barrier()

    ebase = s * EPT

    def _chunk(ch, _):
        base = ebase + ch * CHUNK
        pltpu.sync_copy(col_hbm.at[pl.ds(base, CHUNK)], col_v)
        pltpu.sync_copy(row_hbm.at[pl.ds(base, CHUNK)], row_v)
        pltpu.sync_copy(vals_hbm.at[pl.ds(base, CHUNK)], vals_v)
        pltpu.async_copy(emb_hbm.at[c].at[col_v], rows_v, sem).wait()
        for jo in range(0, CHUNK, 16):
            vblk = vals_v[pl.ds(jo, 16)]
            for ji in range(16):
                j = jo + ji
                vv = vblk[ji]
                for g in range(DH // 16):
                    sl = pl.ds(g * 16, 16)
                    rows_v[j, sl] = rows_v[j, sl] * vv
        pltpu.sync_copy(rows_v, acc.at[row_v], add=True)
        return 0
    lax.fori_loop(0, NCHUNK, _chunk, 0)
    plsc.subcore_barrier()

    for b in range(NRB):
        r0 = s * RPT + b * RBLK
        pltpu.sync_copy(acc.at[pl.ds(r0, RBLK), :], obuf)

        def _lrelu(i, _):
            for g in range(DH // 16):
                sl = pl.ds(g * 16, 16)
                x = obuf[i, sl]
                obuf[i, sl] = jnp.maximum(x, x * 0.5)
            return 0
        lax.fori_loop(0, RBLK, _lrelu, 0)
        pltpu.sync_copy(obuf, out_hbm.at[c, pl.ds(r0, RBLK), :])


def kernel(edge_index, edge_vals, embeds):
    emb_split = embeds.reshape(N_NODES, NC, DH).transpose(1, 0, 2)
    col = edge_index[1]
    row = edge_index[0]

    k = pl.kernel(
        _gcn_body,
        out_type=jax.ShapeDtypeStruct((NC, N_NODES, DH), jnp.float32),
        mesh=plsc.VectorSubcoreMesh(core_axis_name="c", subcore_axis_name="s"),
        compiler_params=pltpu.CompilerParams(use_tc_tiling_on_sc=False),
        scratch_types=[
            pltpu.VMEM((CHUNK,), jnp.int32),
            pltpu.VMEM((CHUNK,), jnp.int32),
            pltpu.VMEM((CHUNK,), jnp.float32),
            pltpu.VMEM((CHUNK, DH), jnp.float32),
            pltpu.VMEM((RBLK, DH), jnp.float32),
            pltpu.VMEM_SHARED((N_NODES, DH), jnp.float32),
            pltpu.SemaphoreType.DMA,
        ],
    )
    out = k(emb_split, col, row, edge_vals)
    return out.transpose(1, 0, 2).reshape(N_NODES, D_FEAT)


# speedup vs baseline: 5.9563x; 2.1415x over previous
import jax
import jax.numpy as jnp
from jax import lax
from jax.experimental import pallas as pl
from jax.experimental.pallas import tpu as pltpu, tpu_sc as plsc

N_NODES = 10000
N_EDGES = 160000
D_FEAT = 256

NC = 2
NS = 16
DH = D_FEAT // NC
EPT = N_EDGES // NS
CHUNK = 80
NCHUNK = EPT // CHUNK
RPT = N_NODES // NS
RBLK = 25
NRB = RPT // RBLK


def _gcn_body(emb_hbm, col_hbm, row_hbm, vals_hbm, out_hbm,
              col_v, vals_v, rowi0, rowi1, rows0, rows1, obuf, acc,
              sem0, sem1):
    c = lax.axis_index("c")
    s = lax.axis_index("s")

    def _zero(i, _):
        for g in range(DH // 16):
            obuf[i, pl.ds(g * 16, 16)] = jnp.zeros((16,), jnp.float32)
        return 0
    lax.fori_loop(0, RBLK, _zero, 0)
    for b in range(NRB):
        pltpu.sync_copy(obuf, acc.at[pl.ds(s * RPT + b * RBLK, RBLK), :])

    ebase = s * EPT
    pltpu.sync_copy(col_hbm.at[pl.ds(ebase, EPT)], col_v)
    pltpu.sync_copy(vals_hbm.at[pl.ds(ebase, EPT)], vals_v)
    plsc.subcore_barrier()

    emb_c = emb_hbm.at[c]

    def _gather(ch, buf, rowi, sem):
        idx = col_v.at[pl.ds(ch * CHUNK, CHUNK)]
        pltpu.async_copy(emb_c.at[idx], buf, sem)
        pltpu.async_copy(row_hbm.at[pl.ds(ebase + ch * CHUNK, CHUNK)],
                         rowi, sem)

    def _wait(buf, rowi, sem):
        pltpu.make_async_copy(emb_c.at[col_v.at[pl.ds(0, CHUNK)]], buf,
                              sem).wait()
        pltpu.make_async_copy(row_hbm.at[pl.ds(0, CHUNK)], rowi, sem).wait()

    def _compute_scatter(ch, buf, rowi):
        for jo in range(0, CHUNK, 16):
            vblk = vals_v[pl.ds(ch * CHUNK + jo, 16)]
            for ji in range(16):
                j = jo + ji
                vv = vblk[ji]
                for g in range(DH // 16):
                    sl = pl.ds(g * 16, 16)
                    buf[j, sl] = buf[j, sl] * vv
        pltpu.sync_copy(buf, acc.at[rowi], add=True)

    _gather(0, rows0, rowi0, sem0)
    _gather(1, rows1, rowi1, sem1)

    def _pair(i, _):
        ch0 = 2 * i
        _wait(rows0, rowi0, sem0)
        _compute_scatter(ch0, rows0, rowi0)
        _gather(ch0 + 2, rows0, rowi0, sem0)
        _wait(rows1, rowi1, sem1)
        _compute_scatter(ch0 + 1, rows1, rowi1)

        @pl.when(i < (NCHUNK - 1) // 2 - 1)
        def _():
            _gather(ch0 + 3, rows1, rowi1, sem1)
        return 0
    lax.fori_loop(0, (NCHUNK - 1) // 2, _pair, 0)

    _wait(rows0, rowi0, sem0)
    _compute_scatter(NCHUNK - 1, rows0, rowi0)
    plsc.subcore_barrier()

    for b in range(NRB):
        r0 = s * RPT + b * RBLK
        pltpu.sync_copy(acc.at[pl.ds(r0, RBLK), :], obuf)

        def _lrelu(i, _):
            for g in range(DH // 16):
                sl = pl.ds(g * 16, 16)
                x = obuf[i, sl]
                obuf[i, sl] = jnp.maximum(x, x * 0.5)
            return 0
        lax.fori_loop(0, RBLK, _lrelu, 0)
        pltpu.sync_copy(obuf, out_hbm.at[c, pl.ds(r0, RBLK), :])


def kernel(edge_index, edge_vals, embeds):
    emb_split = embeds.reshape(N_NODES, NC, DH).transpose(1, 0, 2)
    col = edge_index[1]
    row = edge_index[0]

    k = pl.kernel(
        _gcn_body,
        out_type=jax.ShapeDtypeStruct((NC, N_NODES, DH), jnp.float32),
        mesh=plsc.VectorSubcoreMesh(core_axis_name="c", subcore_axis_name="s"),
        compiler_params=pltpu.CompilerParams(use_tc_tiling_on_sc=False),
        scratch_types=[
            pltpu.VMEM((EPT,), jnp.int32),
            pltpu.VMEM((EPT,), jnp.float32),
            pltpu.VMEM((CHUNK,), jnp.int32),
            pltpu.VMEM((CHUNK,), jnp.int32),
            pltpu.VMEM((CHUNK, DH), jnp.float32),
            pltpu.VMEM((CHUNK, DH), jnp.float32),
            pltpu.VMEM((RBLK, DH), jnp.float32),
            pltpu.VMEM_SHARED((N_NODES, DH), jnp.float32),
            pltpu.SemaphoreType.DMA,
            pltpu.SemaphoreType.DMA,
        ],
    )
    out = k(emb_split, col, row, edge_vals)
    return out.transpose(1, 0, 2).reshape(N_NODES, D_FEAT)
